# baseline (speedup 1.0000x reference)
import functools

import jax
import jax.numpy as jnp
from jax.experimental import pallas as pl
from jax.experimental.pallas import tpu as pltpu


def _moe_body(x_ref, gu_ref, dn_ref, pes_ref, rs_ref, gwt_ref, o_ref, *, E, F, D):
    xt = x_ref[...]
    var = jnp.mean(xt * xt, axis=1, keepdims=True)
    xn = xt * jax.lax.rsqrt(var + 1e-6)
    h = xn * (D ** -0.5) * rs_ref[...]
    logits = jnp.dot(h, gwt_ref[...], preferred_element_type=jnp.float32)
    mx = jnp.max(logits, axis=1, keepdims=True)
    ex = jnp.exp(logits - mx)
    probs = ex / jnp.sum(ex, axis=1, keepdims=True)
    m1 = jnp.max(probs, axis=1, keepdims=True)
    m2 = jnp.max(jnp.where(probs >= m1, -jnp.inf, probs), axis=1, keepdims=True)
    wsel = jnp.where(probs >= m2, probs, 0.0)
    wmask = (wsel / (m1 + m2)) * pes_ref[...]
    acc = jnp.zeros(xt.shape, jnp.float32)
    xb = xt.astype(jnp.bfloat16)
    for e in range(E):
        h2 = jnp.dot(xb, gu_ref[e].astype(jnp.bfloat16),
                     preferred_element_type=jnp.float32)
        gate = h2[:, :F]
        up = h2[:, F:]
        act = 0.5 * gate * (1.0 + jax.lax.erf(gate * (2.0 ** -0.5))) * up
        y = jnp.dot(act.astype(jnp.bfloat16), dn_ref[e].astype(jnp.bfloat16),
                    preferred_element_type=jnp.float32)
        acc = acc + wmask[:, e:e + 1] * y
    o_ref[...] = acc


def kernel(x, gate_up, down, per_expert_scale, router_scale, gate_w):
    B, L, D = x.shape
    E, _, F2 = gate_up.shape
    F = F2 // 2
    N = B * L
    x2 = x.reshape(N, D)
    gate_wT = gate_w.T
    pes = per_expert_scale.reshape(1, E)
    rs = router_scale.reshape(1, D)

    TT = 256
    grid = (N // TT,)
    out = pl.pallas_call(
        functools.partial(_moe_body, E=E, F=F, D=D),
        grid=grid,
        in_specs=[
            pl.BlockSpec((TT, D), lambda i: (i, 0)),
            pl.BlockSpec((E, D, F2), lambda i: (0, 0, 0)),
            pl.BlockSpec((E, F, D), lambda i: (0, 0, 0)),
            pl.BlockSpec((1, E), lambda i: (0, 0)),
            pl.BlockSpec((1, D), lambda i: (0, 0)),
            pl.BlockSpec((D, E), lambda i: (0, 0)),
        ],
        out_specs=pl.BlockSpec((TT, D), lambda i: (i, 0)),
        out_shape=jax.ShapeDtypeStruct((N, D), jnp.float32),
        compiler_params=pltpu.CompilerParams(
            dimension_semantics=("arbitrary",),
            vmem_limit_bytes=100 * 1024 * 1024,
        ),
    )(x2, gate_up, down, pes, rs, gate_wT)
    return out.reshape(B, L, D)


# speedup vs baseline: 1.2182x; 1.2182x over previous
import functools

import jax
import jax.numpy as jnp
from jax.experimental import pallas as pl
from jax.experimental.pallas import tpu as pltpu


def _moe_body(x_ref, gu_ref, dn_ref, pes_ref, rs_ref, gwt_ref, o_ref, *, E, F, D):
    xt = x_ref[...]
    var = jnp.mean(xt * xt, axis=1, keepdims=True)
    xn = xt * jax.lax.rsqrt(var + 1e-6)
    h = xn * (D ** -0.5) * rs_ref[...]
    logits = jnp.dot(h, gwt_ref[...], preferred_element_type=jnp.float32)
    mx = jnp.max(logits, axis=1, keepdims=True)
    ex = jnp.exp(logits - mx)
    probs = ex / jnp.sum(ex, axis=1, keepdims=True)
    m1 = jnp.max(probs, axis=1, keepdims=True)
    m2 = jnp.max(jnp.where(probs >= m1, -jnp.inf, probs), axis=1, keepdims=True)
    wsel = jnp.where(probs >= m2, probs, 0.0)
    wmask = (wsel / (m1 + m2)) * pes_ref[...]
    acc = jnp.zeros(xt.shape, jnp.float32)
    for e in range(E):
        h2 = jnp.dot(xt, gu_ref[e], preferred_element_type=jnp.float32)
        gate = h2[:, :F]
        up = h2[:, F:]
        act = 0.5 * gate * (1.0 + jax.lax.erf(gate * (2.0 ** -0.5))) * up
        y = jnp.dot(act, dn_ref[e], preferred_element_type=jnp.float32)
        acc = acc + wmask[:, e:e + 1] * y
    o_ref[...] = acc


def kernel(x, gate_up, down, per_expert_scale, router_scale, gate_w):
    B, L, D = x.shape
    E, _, F2 = gate_up.shape
    F = F2 // 2
    N = B * L
    x2 = x.reshape(N, D)
    gate_wT = gate_w.T
    pes = per_expert_scale.reshape(1, E)
    rs = router_scale.reshape(1, D)

    TT = 512
    grid = (N // TT,)
    out = pl.pallas_call(
        functools.partial(_moe_body, E=E, F=F, D=D),
        grid=grid,
        in_specs=[
            pl.BlockSpec((TT, D), lambda i: (i, 0)),
            pl.BlockSpec((E, D, F2), lambda i: (0, 0, 0)),
            pl.BlockSpec((E, F, D), lambda i: (0, 0, 0)),
            pl.BlockSpec((1, E), lambda i: (0, 0)),
            pl.BlockSpec((1, D), lambda i: (0, 0)),
            pl.BlockSpec((D, E), lambda i: (0, 0)),
        ],
        out_specs=pl.BlockSpec((TT, D), lambda i: (i, 0)),
        out_shape=jax.ShapeDtypeStruct((N, D), jnp.float32),
        compiler_params=pltpu.CompilerParams(
            dimension_semantics=("arbitrary",),
            vmem_limit_bytes=100 * 1024 * 1024,
        ),
    )(x2, gate_up, down, pes, rs, gate_wT)
    return out.reshape(B, L, D)


# speedup vs baseline: 1.2875x; 1.0569x over previous
import functools

import jax
import jax.numpy as jnp
from jax.experimental import pallas as pl
from jax.experimental.pallas import tpu as pltpu


def _moe_body(x_ref, gu_ref, dn_ref, pes_ref, rs_ref, gwt_ref, o_ref, *, E, F, D):
    xt = x_ref[...]
    var = jnp.mean(xt * xt, axis=1, keepdims=True)
    xn = xt * jax.lax.rsqrt(var + 1e-6)
    h = xn * (D ** -0.5) * rs_ref[...]
    logits = jnp.dot(h, gwt_ref[...], preferred_element_type=jnp.float32)
    mx = jnp.max(logits, axis=1, keepdims=True)
    ex = jnp.exp(logits - mx)
    probs = ex / jnp.sum(ex, axis=1, keepdims=True)
    m1 = jnp.max(probs, axis=1, keepdims=True)
    m2 = jnp.max(jnp.where(probs >= m1, -jnp.inf, probs), axis=1, keepdims=True)
    wsel = jnp.where(probs >= m2, probs, 0.0)
    wmask = (wsel / (m1 + m2)) * pes_ref[...]
    acc = jnp.zeros(xt.shape, jnp.float32)
    for e in range(E):
        h2 = jnp.dot(xt, gu_ref[e], preferred_element_type=jnp.float32)
        gate = h2[:, :F]
        up = h2[:, F:]
        act = 0.5 * gate * (1.0 + jax.lax.erf(gate * (2.0 ** -0.5))) * up
        y = jnp.dot(act, dn_ref[e], preferred_element_type=jnp.float32)
        acc = acc + wmask[:, e:e + 1] * y
    o_ref[...] = acc


def kernel(x, gate_up, down, per_expert_scale, router_scale, gate_w):
    B, L, D = x.shape
    E, _, F2 = gate_up.shape
    F = F2 // 2
    N = B * L
    x2 = x.reshape(N, D)
    gate_wT = gate_w.T
    pes = per_expert_scale.reshape(1, E)
    rs = router_scale.reshape(1, D)

    TT = 1024
    grid = (N // TT,)
    out = pl.pallas_call(
        functools.partial(_moe_body, E=E, F=F, D=D),
        grid=grid,
        in_specs=[
            pl.BlockSpec((TT, D), lambda i: (i, 0)),
            pl.BlockSpec((E, D, F2), lambda i: (0, 0, 0)),
            pl.BlockSpec((E, F, D), lambda i: (0, 0, 0)),
            pl.BlockSpec((1, E), lambda i: (0, 0)),
            pl.BlockSpec((1, D), lambda i: (0, 0)),
            pl.BlockSpec((D, E), lambda i: (0, 0)),
        ],
        out_specs=pl.BlockSpec((TT, D), lambda i: (i, 0)),
        out_shape=jax.ShapeDtypeStruct((N, D), jnp.float32),
        compiler_params=pltpu.CompilerParams(
            dimension_semantics=("arbitrary",),
            vmem_limit_bytes=100 * 1024 * 1024,
        ),
    )(x2, gate_up, down, pes, rs, gate_wT)
    return out.reshape(B, L, D)
